# baseline (speedup 1.0000x reference)
import functools

import jax
import jax.numpy as jnp
from jax import lax
from jax.experimental import pallas as pl
from jax.experimental.pallas import tpu as pltpu
from jax.experimental.pallas import tpu_sc as plsc

_VOCAB = 100000
_DIM = 64
_B = 16384
_L = 50
_NW = 32
_BW = _B // _NW
_BT = _BW // 128


def _make_gather():
    mesh = plsc.VectorSubcoreMesh(core_axis_name="c", subcore_axis_name="s")

    @functools.partial(
        pl.kernel,
        mesh=mesh,
        out_type=jax.ShapeDtypeStruct((_L, 8, _B // 128, 8, 128), jnp.float32),
        scratch_types=[
            pltpu.VMEM((_BW,), jnp.int32),
            pltpu.VMEM((_BW,), jnp.int32),
            pltpu.VMEM((_BW, _DIM), jnp.float32),
            pltpu.VMEM((_BW, _DIM), jnp.float32),
            pltpu.VMEM((4, _BT, 8, 128), jnp.float32),
            pltpu.VMEM((4, _BT, 8, 128), jnp.float32),
            pltpu.SemaphoreType.DMA,
            pltpu.SemaphoreType.DMA,
            pltpu.SemaphoreType.DMA,
            pltpu.SemaphoreType.DMA,
            pltpu.SemaphoreType.DMA,
            pltpu.SemaphoreType.DMA,
        ],
        compiler_params=pltpu.CompilerParams(
            use_tc_tiling_on_sc=False, needs_layout_passes=False),
    )
    def gather_kernel(table_hbm, idx_hbm, out_hbm,
                      ilist0, ilist1, staged0, staged1, ttile0, ttile1,
                      isem0, isem1, gsem0, gsem1, osem0, osem1):
        ilists = (ilist0, ilist1)
        stageds = (staged0, staged1)
        ttiles = (ttile0, ttile1)
        isems = (isem0, isem1)
        gsems = (gsem0, gsem1)
        osems = (osem0, osem1)
        wid = lax.axis_index("s") * 2 + lax.axis_index("c")
        b0 = wid * _BW
        bt0 = wid * _BT
        lanes = lax.iota(jnp.int32, 16)

        def load_ilist(l, p):
            pltpu.async_copy(idx_hbm.at[l, pl.ds(b0, _BW)], ilists[p], isems[p])

        def transpose_half(l, p, h):
            staged = stageds[p]
            ttile = ttiles[h]

            @plsc.parallel_loop(0, 32, unroll=4)
            def body(i):
                btl = i >> 3
                j = i & 7
                row = lanes + i * 16
                for dd in range(32):
                    col = jnp.full((16,), 32 * h + dd, jnp.int32)
                    v = plsc.load_gather(staged, [row, col])
                    ttile[dd >> 3, btl, dd & 7, pl.ds(j * 16, 16)] = v
            pltpu.async_copy(
                ttile, out_hbm.at[l, pl.ds(4 * h, 4), pl.ds(bt0, _BT)], osems[h])

        def drain_half(l, h):
            pltpu.make_async_copy(
                ttiles[h], out_hbm.at[l, pl.ds(4 * h, 4), pl.ds(bt0, _BT)],
                osems[h]).wait()

        load_ilist(0, 0)
        pltpu.make_async_copy(idx_hbm.at[0, pl.ds(b0, _BW)], ilist0, isem0).wait()
        pltpu.async_copy(table_hbm.at[ilist0], staged0, gsem0)
        load_ilist(1, 1)

        def group(g, carry):
            for p in range(2):
                l = g * 2 + p
                pltpu.make_async_copy(table_hbm.at[ilists[p]], stageds[p],
                                      gsems[p]).wait()

                @pl.when(l + 2 < _L)
                def _():
                    load_ilist(l + 2, p)

                @pl.when(l + 1 < _L)
                def _():
                    pltpu.make_async_copy(idx_hbm.at[l + 1, pl.ds(b0, _BW)],
                                          ilists[1 - p], isems[1 - p]).wait()
                    pltpu.async_copy(table_hbm.at[ilists[1 - p]],
                                     stageds[1 - p], gsems[1 - p])

                for h in range(2):
                    @pl.when(l >= 1)
                    def _():
                        drain_half(l - 1, h)
                    transpose_half(l, p, h)
            return carry

        lax.fori_loop(0, _L // 2, group, 0)

        for h in range(2):
            drain_half(_L - 1, h)

    return gather_kernel


_gather = _make_gather()


@jax.jit
def kernel(input, embd_weight):
    idx_lb = input.T.astype(jnp.int32)
    out6 = _gather(embd_weight, idx_lb)
    return jnp.transpose(out6, (2, 4, 0, 1, 3)).reshape(_B, _L, _DIM)


# speedup vs baseline: 1.0586x; 1.0586x over previous
import functools

import jax
import jax.numpy as jnp
from jax import lax
from jax.experimental import pallas as pl
from jax.experimental.pallas import tpu as pltpu
from jax.experimental.pallas import tpu_sc as plsc

_VOCAB = 100000
_DIM = 64
_B = 16384
_L = 50
_NW = 32
_BW = _B // _NW
_BT = _BW // 128


def _make_gather():
    mesh = plsc.VectorSubcoreMesh(core_axis_name="c", subcore_axis_name="s")

    @functools.partial(
        pl.kernel,
        mesh=mesh,
        out_type=jax.ShapeDtypeStruct((_L, 8, _B // 128, 8, 128), jnp.float32),
        scratch_types=[
            pltpu.VMEM((_BW,), jnp.int32),
            pltpu.VMEM((_BW,), jnp.int32),
            pltpu.VMEM((_BW, _DIM), jnp.float32),
            pltpu.VMEM((_BW, _DIM), jnp.float32),
            pltpu.VMEM((4, _BT, 8, 128), jnp.float32),
            pltpu.VMEM((4, _BT, 8, 128), jnp.float32),
            pltpu.SemaphoreType.DMA,
            pltpu.SemaphoreType.DMA,
            pltpu.SemaphoreType.DMA,
            pltpu.SemaphoreType.DMA,
            pltpu.SemaphoreType.DMA,
            pltpu.SemaphoreType.DMA,
        ],
        compiler_params=pltpu.CompilerParams(
            use_tc_tiling_on_sc=False, needs_layout_passes=False),
    )
    def gather_kernel(table_hbm, idx_hbm, out_hbm,
                      ilist0, ilist1, staged0, staged1, ttile0, ttile1,
                      isem0, isem1, gsem0, gsem1, osem0, osem1):
        ilists = (ilist0, ilist1)
        stageds = (staged0, staged1)
        ttiles = (ttile0, ttile1)
        isems = (isem0, isem1)
        gsems = (gsem0, gsem1)
        osems = (osem0, osem1)
        wid = lax.axis_index("s") * 2 + lax.axis_index("c")
        b0 = wid * _BW
        bt0 = wid * _BT
        lanes = lax.iota(jnp.int32, 16)

        def load_ilist(l, p):
            pltpu.async_copy(idx_hbm.at[l, pl.ds(b0, _BW)], ilists[p], isems[p])

        def transpose_half(l, p, h):
            staged = stageds[p]
            ttile = ttiles[h]

            @plsc.parallel_loop(0, 32, unroll=8)
            def body(d):
                dt = d >> 3
                ds = d & 7
                col = jnp.full((16,), 32 * h, jnp.int32) + d
                for btl in range(_BT):
                    for j in range(8):
                        row = lanes + (btl * 128 + j * 16)
                        v = plsc.load_gather(staged, [row, col])
                        ttile[dt, btl, ds, pl.ds(j * 16, 16)] = v
            pltpu.async_copy(
                ttile, out_hbm.at[l, pl.ds(4 * h, 4), pl.ds(bt0, _BT)], osems[h])

        def drain_half(l, h):
            pltpu.make_async_copy(
                ttiles[h], out_hbm.at[l, pl.ds(4 * h, 4), pl.ds(bt0, _BT)],
                osems[h]).wait()

        load_ilist(0, 0)
        pltpu.make_async_copy(idx_hbm.at[0, pl.ds(b0, _BW)], ilist0, isem0).wait()
        pltpu.async_copy(table_hbm.at[ilist0], staged0, gsem0)
        load_ilist(1, 1)

        def group(g, carry):
            for p in range(2):
                l = g * 2 + p
                pltpu.make_async_copy(table_hbm.at[ilists[p]], stageds[p],
                                      gsems[p]).wait()

                @pl.when(l + 2 < _L)
                def _():
                    load_ilist(l + 2, p)

                @pl.when(l + 1 < _L)
                def _():
                    pltpu.make_async_copy(idx_hbm.at[l + 1, pl.ds(b0, _BW)],
                                          ilists[1 - p], isems[1 - p]).wait()
                    pltpu.async_copy(table_hbm.at[ilists[1 - p]],
                                     stageds[1 - p], gsems[1 - p])

                for h in range(2):
                    @pl.when(l >= 1)
                    def _():
                        drain_half(l - 1, h)
                    transpose_half(l, p, h)
            return carry

        lax.fori_loop(0, _L // 2, group, 0)

        for h in range(2):
            drain_half(_L - 1, h)

    return gather_kernel


_gather = _make_gather()


@jax.jit
def kernel(input, embd_weight):
    idx_lb = input.T.astype(jnp.int32)
    out6 = _gather(embd_weight, idx_lb)
    return jnp.transpose(out6, (2, 4, 0, 1, 3)).reshape(_B, _L, _DIM)


# speedup vs baseline: 1.2394x; 1.1709x over previous
import functools

import jax
import jax.numpy as jnp
from jax import lax
from jax.experimental import pallas as pl
from jax.experimental.pallas import tpu as pltpu
from jax.experimental.pallas import tpu_sc as plsc

_VOCAB = 100000
_DIM = 64
_B = 16384
_L = 50
_NW = 32
_BW = _B // _NW
_BT = _BW // 128


def _make_gather():
    mesh = plsc.VectorSubcoreMesh(core_axis_name="c", subcore_axis_name="s")

    @functools.partial(
        pl.kernel,
        mesh=mesh,
        out_type=jax.ShapeDtypeStruct((_L, 8, _B // 128, 8, 128), jnp.float32),
        scratch_types=[
            pltpu.VMEM((_BW,), jnp.int32),
            pltpu.VMEM((_BW,), jnp.int32),
            pltpu.VMEM((_BW, _DIM), jnp.float32),
            pltpu.VMEM((_BW, _DIM), jnp.float32),
            pltpu.VMEM((4, _BT, 8, 128), jnp.float32),
            pltpu.VMEM((4, _BT, 8, 128), jnp.float32),
            pltpu.SemaphoreType.DMA,
            pltpu.SemaphoreType.DMA,
            pltpu.SemaphoreType.DMA,
            pltpu.SemaphoreType.DMA,
            pltpu.SemaphoreType.DMA,
            pltpu.SemaphoreType.DMA,
        ],
        compiler_params=pltpu.CompilerParams(
            use_tc_tiling_on_sc=False, needs_layout_passes=False,
            disable_bounds_checks=True),
    )
    def gather_kernel(table_hbm, idx_hbm, out_hbm,
                      ilist0, ilist1, staged0, staged1, ttile0, ttile1,
                      isem0, isem1, gsem0, gsem1, osem0, osem1):
        ilists = (ilist0, ilist1)
        stageds = (staged0, staged1)
        ttiles = (ttile0, ttile1)
        isems = (isem0, isem1)
        gsems = (gsem0, gsem1)
        osems = (osem0, osem1)
        wid = lax.axis_index("s") * 2 + lax.axis_index("c")
        b0 = wid * _BW
        bt0 = wid * _BT
        lanes = lax.iota(jnp.int32, 16)

        def load_ilist(l, p):
            pltpu.async_copy(idx_hbm.at[l, pl.ds(b0, _BW)], ilists[p], isems[p])

        def transpose_half(l, p, h):
            staged = stageds[p]
            ttile = ttiles[h]

            @plsc.parallel_loop(0, 32, unroll=4)
            def body(d):
                dt = d >> 3
                ds = d & 7
                col = jnp.full((16,), 32 * h, jnp.int32) + d
                for btl in range(_BT):
                    for j in range(8):
                        row = lanes + (btl * 128 + j * 16)
                        v = plsc.load_gather(staged, [row, col])
                        ttile[dt, btl, ds, pl.ds(j * 16, 16)] = v
            pltpu.async_copy(
                ttile, out_hbm.at[l, pl.ds(4 * h, 4), pl.ds(bt0, _BT)], osems[h])

        def drain_half(l, h):
            pltpu.make_async_copy(
                ttiles[h], out_hbm.at[l, pl.ds(4 * h, 4), pl.ds(bt0, _BT)],
                osems[h]).wait()

        load_ilist(0, 0)
        pltpu.make_async_copy(idx_hbm.at[0, pl.ds(b0, _BW)], ilist0, isem0).wait()
        pltpu.async_copy(table_hbm.at[ilist0], staged0, gsem0)
        load_ilist(1, 1)

        def group(g, carry):
            for p in range(2):
                l = g * 2 + p
                pltpu.make_async_copy(table_hbm.at[ilists[p]], stageds[p],
                                      gsems[p]).wait()

                @pl.when(l + 2 < _L)
                def _():
                    load_ilist(l + 2, p)

                @pl.when(l + 1 < _L)
                def _():
                    pltpu.make_async_copy(idx_hbm.at[l + 1, pl.ds(b0, _BW)],
                                          ilists[1 - p], isems[1 - p]).wait()
                    pltpu.async_copy(table_hbm.at[ilists[1 - p]],
                                     stageds[1 - p], gsems[1 - p])

                for h in range(2):
                    @pl.when(l >= 1)
                    def _():
                        drain_half(l - 1, h)
                    transpose_half(l, p, h)
            return carry

        lax.fori_loop(0, _L // 2, group, 0)

        for h in range(2):
            drain_half(_L - 1, h)

    return gather_kernel


_gather = _make_gather()


@jax.jit
def kernel(input, embd_weight):
    idx_lb = input.T.astype(jnp.int32)
    out6 = _gather(embd_weight, idx_lb)
    return jnp.transpose(out6, (2, 4, 0, 1, 3)).reshape(_B, _L, _DIM)


# speedup vs baseline: 3.0169x; 2.4341x over previous
import functools

import jax
import jax.numpy as jnp
from jax import lax
from jax.experimental import pallas as pl
from jax.experimental.pallas import tpu as pltpu
from jax.experimental.pallas import tpu_sc as plsc

_VOCAB = 100000
_DIM = 64
_B = 16384
_L = 50
_NW = 32
_BW = _B // _NW
_BT = _BW // 128
_PADD = 65


def _make_gather():
    mesh = plsc.VectorSubcoreMesh(core_axis_name="c", subcore_axis_name="s")

    @functools.partial(
        pl.kernel,
        mesh=mesh,
        out_type=jax.ShapeDtypeStruct((_L, 8, _B // 128, 8, 128), jnp.float32),
        scratch_types=[
            pltpu.VMEM((_BW,), jnp.int32),
            pltpu.VMEM((_BW,), jnp.int32),
            pltpu.VMEM((_BW, _PADD), jnp.float32),
            pltpu.VMEM((_BW, _PADD), jnp.float32),
            pltpu.VMEM((4, _BT, 8, 128), jnp.float32),
            pltpu.VMEM((4, _BT, 8, 128), jnp.float32),
            pltpu.SemaphoreType.DMA,
            pltpu.SemaphoreType.DMA,
            pltpu.SemaphoreType.DMA,
            pltpu.SemaphoreType.DMA,
            pltpu.SemaphoreType.DMA,
            pltpu.SemaphoreType.DMA,
        ],
        compiler_params=pltpu.CompilerParams(
            use_tc_tiling_on_sc=False, needs_layout_passes=False,
            disable_bounds_checks=True),
    )
    def gather_kernel(table_hbm, idx_hbm, out_hbm,
                      ilist0, ilist1, staged0, staged1, ttile0, ttile1,
                      isem0, isem1, gsem0, gsem1, osem0, osem1):
        ilists = (ilist0, ilist1)
        stageds = (staged0, staged1)
        ttiles = (ttile0, ttile1)
        isems = (isem0, isem1)
        gsems = (gsem0, gsem1)
        osems = (osem0, osem1)
        wid = lax.axis_index("s") * 2 + lax.axis_index("c")
        b0 = wid * _BW
        bt0 = wid * _BT
        lanes = lax.iota(jnp.int32, 16)

        def load_ilist(l, p):
            pltpu.async_copy(idx_hbm.at[l, pl.ds(b0, _BW)], ilists[p], isems[p])

        def transpose_half(l, p, h):
            staged = stageds[p]
            ttile = ttiles[h]

            @plsc.parallel_loop(0, 32, unroll=4)
            def body(d):
                dt = d >> 3
                ds = d & 7
                col = jnp.full((16,), 32 * h, jnp.int32) + d
                for btl in range(_BT):
                    for j in range(8):
                        row = lanes + (btl * 128 + j * 16)
                        v = plsc.load_gather(staged, [row, col])
                        ttile[dt, btl, ds, pl.ds(j * 16, 16)] = v
            pltpu.async_copy(
                ttile, out_hbm.at[l, pl.ds(4 * h, 4), pl.ds(bt0, _BT)], osems[h])

        def drain_half(l, h):
            pltpu.make_async_copy(
                ttiles[h], out_hbm.at[l, pl.ds(4 * h, 4), pl.ds(bt0, _BT)],
                osems[h]).wait()

        load_ilist(0, 0)
        pltpu.make_async_copy(idx_hbm.at[0, pl.ds(b0, _BW)], ilist0, isem0).wait()
        pltpu.async_copy(table_hbm.at[ilist0], staged0, gsem0)
        load_ilist(1, 1)

        def group(g, carry):
            for p in range(2):
                l = g * 2 + p
                pltpu.make_async_copy(table_hbm.at[ilists[p]], stageds[p],
                                      gsems[p]).wait()

                @pl.when(l + 2 < _L)
                def _():
                    load_ilist(l + 2, p)

                @pl.when(l + 1 < _L)
                def _():
                    pltpu.make_async_copy(idx_hbm.at[l + 1, pl.ds(b0, _BW)],
                                          ilists[1 - p], isems[1 - p]).wait()
                    pltpu.async_copy(table_hbm.at[ilists[1 - p]],
                                     stageds[1 - p], gsems[1 - p])

                for h in range(2):
                    @pl.when(l >= 1)
                    def _():
                        drain_half(l - 1, h)
                    transpose_half(l, p, h)
            return carry

        lax.fori_loop(0, _L // 2, group, 0)

        for h in range(2):
            drain_half(_L - 1, h)

    return gather_kernel


_gather = _make_gather()


@jax.jit
def kernel(input, embd_weight):
    idx_lb = input.T.astype(jnp.int32)
    table_p = jnp.pad(embd_weight, ((0, 0), (0, _PADD - _DIM)))
    out6 = _gather(table_p, idx_lb)
    return jnp.transpose(out6, (2, 4, 0, 1, 3)).reshape(_B, _L, _DIM)


# speedup vs baseline: 3.0183x; 1.0005x over previous
import functools

import jax
import jax.numpy as jnp
from jax import lax
from jax.experimental import pallas as pl
from jax.experimental.pallas import tpu as pltpu
from jax.experimental.pallas import tpu_sc as plsc

_VOCAB = 100000
_DIM = 64
_B = 16384
_L = 50
_NW = 32
_BW = _B // _NW
_BT = _BW // 128
_PADD = 72


def _make_gather():
    mesh = plsc.VectorSubcoreMesh(core_axis_name="c", subcore_axis_name="s")

    @functools.partial(
        pl.kernel,
        mesh=mesh,
        out_type=jax.ShapeDtypeStruct((_L, 8, _B // 128, 8, 128), jnp.float32),
        scratch_types=[
            pltpu.VMEM((_BW,), jnp.int32),
            pltpu.VMEM((_BW,), jnp.int32),
            pltpu.VMEM((_BW, _PADD), jnp.float32),
            pltpu.VMEM((_BW, _PADD), jnp.float32),
            pltpu.VMEM((4, _BT, 8, 128), jnp.float32),
            pltpu.VMEM((4, _BT, 8, 128), jnp.float32),
            pltpu.SemaphoreType.DMA,
            pltpu.SemaphoreType.DMA,
            pltpu.SemaphoreType.DMA,
            pltpu.SemaphoreType.DMA,
            pltpu.SemaphoreType.DMA,
            pltpu.SemaphoreType.DMA,
        ],
        compiler_params=pltpu.CompilerParams(
            use_tc_tiling_on_sc=False, needs_layout_passes=False,
            disable_bounds_checks=True),
    )
    def gather_kernel(table_hbm, idx_hbm, out_hbm,
                      ilist0, ilist1, staged0, staged1, ttile0, ttile1,
                      isem0, isem1, gsem0, gsem1, osem0, osem1):
        ilists = (ilist0, ilist1)
        stageds = (staged0, staged1)
        ttiles = (ttile0, ttile1)
        isems = (isem0, isem1)
        gsems = (gsem0, gsem1)
        osems = (osem0, osem1)
        wid = lax.axis_index("s") * 2 + lax.axis_index("c")
        b0 = wid * _BW
        bt0 = wid * _BT
        lanes = lax.iota(jnp.int32, 16)

        def load_ilist(l, p):
            pltpu.async_copy(idx_hbm.at[l, pl.ds(b0, _BW)], ilists[p], isems[p])

        def transpose_half(l, p, h):
            staged = stageds[p]
            ttile = ttiles[h]

            @plsc.parallel_loop(0, 32, unroll=4)
            def body(d):
                dt = d >> 3
                ds = d & 7
                col = jnp.full((16,), 32 * h, jnp.int32) + d
                for btl in range(_BT):
                    for j in range(8):
                        row = lanes + (btl * 128 + j * 16)
                        v = plsc.load_gather(staged, [row, col])
                        ttile[dt, btl, ds, pl.ds(j * 16, 16)] = v
            pltpu.async_copy(
                ttile, out_hbm.at[l, pl.ds(4 * h, 4), pl.ds(bt0, _BT)], osems[h])

        def drain_half(l, h):
            pltpu.make_async_copy(
                ttiles[h], out_hbm.at[l, pl.ds(4 * h, 4), pl.ds(bt0, _BT)],
                osems[h]).wait()

        load_ilist(0, 0)
        pltpu.make_async_copy(idx_hbm.at[0, pl.ds(b0, _BW)], ilist0, isem0).wait()
        pltpu.async_copy(table_hbm.at[ilist0], staged0, gsem0)
        load_ilist(1, 1)

        def group(g, carry):
            for p in range(2):
                l = g * 2 + p
                pltpu.make_async_copy(table_hbm.at[ilists[p]], stageds[p],
                                      gsems[p]).wait()

                @pl.when(l + 2 < _L)
                def _():
                    load_ilist(l + 2, p)

                @pl.when(l + 1 < _L)
                def _():
                    pltpu.make_async_copy(idx_hbm.at[l + 1, pl.ds(b0, _BW)],
                                          ilists[1 - p], isems[1 - p]).wait()
                    pltpu.async_copy(table_hbm.at[ilists[1 - p]],
                                     stageds[1 - p], gsems[1 - p])

                for h in range(2):
                    @pl.when(l >= 1)
                    def _():
                        drain_half(l - 1, h)
                    transpose_half(l, p, h)
            return carry

        lax.fori_loop(0, _L // 2, group, 0)

        for h in range(2):
            drain_half(_L - 1, h)

    return gather_kernel


_gather = _make_gather()


@jax.jit
def kernel(input, embd_weight):
    idx_lb = input.T.astype(jnp.int32)
    table_p = jnp.pad(embd_weight, ((0, 0), (0, _PADD - _DIM)))
    out6 = _gather(table_p, idx_lb)
    return jnp.transpose(out6, (2, 4, 0, 1, 3)).reshape(_B, _L, _DIM)


# speedup vs baseline: 4.0058x; 1.3271x over previous
import functools

import jax
import jax.numpy as jnp
from jax import lax
from jax.experimental import pallas as pl
from jax.experimental.pallas import tpu as pltpu
from jax.experimental.pallas import tpu_sc as plsc

_VOCAB = 100000
_DIM = 64
_B = 16384
_L = 50
_NW = 32
_BW = _B // _NW
_BT = _BW // 128
_BSP = 129


def _make_gather():
    mesh = plsc.VectorSubcoreMesh(core_axis_name="c", subcore_axis_name="s")

    @functools.partial(
        pl.kernel,
        mesh=mesh,
        out_type=jax.ShapeDtypeStruct((_L, 8, _B // 128, 8, 128), jnp.float32),
        scratch_types=[
            pltpu.VMEM((_BW,), jnp.int32),
            pltpu.VMEM((_BW,), jnp.int32),
            pltpu.VMEM((_BW, _DIM), jnp.float32),
            pltpu.VMEM((_BW, _DIM), jnp.float32),
            pltpu.VMEM((2, _DIM, _BSP), jnp.float32),
            pltpu.VMEM((2, _DIM, _BSP), jnp.float32),
            pltpu.SemaphoreType.DMA,
            pltpu.SemaphoreType.DMA,
            pltpu.SemaphoreType.DMA,
            pltpu.SemaphoreType.DMA,
            pltpu.SemaphoreType.DMA,
            pltpu.SemaphoreType.DMA,
        ],
        compiler_params=pltpu.CompilerParams(
            use_tc_tiling_on_sc=False, needs_layout_passes=False,
            disable_bounds_checks=True),
    )
    def gather_kernel(table_hbm, idx_hbm, out_hbm,
                      ilist0, ilist1, staged0, staged1, ttile0, ttile1,
                      isem0, isem1, gsem0, gsem1, osem0, osem1):
        ilists = (ilist0, ilist1)
        stageds = (staged0, staged1)
        ttiles = (ttile0, ttile1)
        isems = (isem0, isem1)
        gsems = (gsem0, gsem1)
        osems = (osem0, osem1)
        wid = lax.axis_index("s") * 2 + lax.axis_index("c")
        b0 = wid * _BW
        bt0 = wid * _BT
        lanes = lax.iota(jnp.int32, 16)

        def load_ilist(l, p):
            pltpu.async_copy(idx_hbm.at[l, pl.ds(b0, _BW)], ilists[p], isems[p])

        def transpose_half(l, p, h):
            staged = stageds[p]
            ttile = ttiles[h]

            @plsc.parallel_loop(0, 128, unroll=4)
            def body(bs):
                bsv = jnp.full((16,), 0, jnp.int32) + bs
                for btl2 in range(2):
                    row = 256 * h + btl2 * 128 + bs
                    btv = jnp.full((16,), btl2, jnp.int32)
                    for d0 in range(0, _DIM, 16):
                        v = staged[row, pl.ds(d0, 16)]
                        plsc.store_scatter(ttile, [btv, d0 + lanes, bsv], v)

            for dt in range(8):
                pltpu.async_copy(
                    ttile.at[:, pl.ds(dt * 8, 8), pl.ds(0, 128)],
                    out_hbm.at[l, dt, pl.ds(bt0 + 2 * h, 2)],
                    osems[h])

        def drain_half(l, h):
            for dt in range(8):
                pltpu.make_async_copy(
                    ttiles[h].at[:, pl.ds(dt * 8, 8), pl.ds(0, 128)],
                    out_hbm.at[l, dt, pl.ds(bt0 + 2 * h, 2)],
                    osems[h]).wait()

        load_ilist(0, 0)
        pltpu.make_async_copy(idx_hbm.at[0, pl.ds(b0, _BW)], ilist0, isem0).wait()
        pltpu.async_copy(table_hbm.at[ilist0], staged0, gsem0)
        load_ilist(1, 1)

        def group(g, carry):
            for p in range(2):
                l = g * 2 + p
                pltpu.make_async_copy(table_hbm.at[ilists[p]], stageds[p],
                                      gsems[p]).wait()

                @pl.when(l + 2 < _L)
                def _():
                    load_ilist(l + 2, p)

                @pl.when(l + 1 < _L)
                def _():
                    pltpu.make_async_copy(idx_hbm.at[l + 1, pl.ds(b0, _BW)],
                                          ilists[1 - p], isems[1 - p]).wait()
                    pltpu.async_copy(table_hbm.at[ilists[1 - p]],
                                     stageds[1 - p], gsems[1 - p])

                for h in range(2):
                    @pl.when(l >= 1)
                    def _():
                        drain_half(l - 1, h)
                    transpose_half(l, p, h)
            return carry

        lax.fori_loop(0, _L // 2, group, 0)

        for h in range(2):
            drain_half(_L - 1, h)

    return gather_kernel


_gather = _make_gather()


@jax.jit
def kernel(input, embd_weight):
    idx_lb = input.T.astype(jnp.int32)
    out6 = _gather(embd_weight, idx_lb)
    return jnp.transpose(out6, (2, 4, 0, 1, 3)).reshape(_B, _L, _DIM)
